# baseline (speedup 1.0000x reference)
import jax
import jax.numpy as jnp
from jax import lax
from jax.experimental import pallas as pl
from jax.experimental.pallas import tpu as pltpu
from jax.experimental.pallas import tpu_sc as plsc

NC = 2
NS = 16
NW = NC * NS
L = 16

N = 10000
E = 320000
H = 128
HB = H // L

PN = 320
NPAD = NW * PN

EPW = E // NW
CB = 200
NCH_A = EPW // CB
SUBG = ((0, 128), (128, CB - 128))

CH = 1280
NCH_C = E // CH
NST = CH // L
SROWS = CH // NS
Q = 64


def _gather_body(x_hbm, src_hbm, dst_hbm, w_hbm, a1_hbm, a2_hbm,
                 sb0, db0, wb0, xi0, xj0, sb1, db1, wb1, xi1, xj1,
                 si0, si1, sg0, sg1, sw0, sw1):
  wid = lax.axis_index("s") * NC + lax.axis_index("c")
  base = wid * EPW

  bufs = ((sb0, db0, wb0, xi0, xj0, si0, sg0, sw0),
          (sb1, db1, wb1, xi1, xj1, si1, sg1, sw1))

  def fire_ids(ci, b):
    sb, db, wb, _, _, si, _, _ = bufs[b]
    off = base + ci * CB
    pltpu.make_async_copy(src_hbm.at[pl.ds(off, CB)], sb, si).start()
    pltpu.make_async_copy(dst_hbm.at[pl.ds(off, CB)], db, si).start()
    pltpu.make_async_copy(
        w_hbm.at[pl.ds(off, CB)], wb.at[pl.ds(0, CB)], si).start()

  def wait_ids(b):
    sb, db, wb, _, _, si, _, _ = bufs[b]
    pltpu.make_async_copy(src_hbm.at[pl.ds(0, CB)], sb, si).wait()
    pltpu.make_async_copy(dst_hbm.at[pl.ds(0, CB)], db, si).wait()
    pltpu.make_async_copy(
        w_hbm.at[pl.ds(0, CB)], wb.at[pl.ds(0, CB)], si).wait()

  def fire_gathers(b):
    sb, db, _, xi, xj, _, sg, _ = bufs[b]
    for (o, n) in SUBG:
      pltpu.make_async_copy(
          x_hbm.at[db.at[pl.ds(o, n)]], xi.at[pl.ds(o, n)], sg).start()
      pltpu.make_async_copy(
          x_hbm.at[sb.at[pl.ds(o, n)]], xj.at[pl.ds(o, n)], sg).start()

  def wait_gathers(b):
    sb, db, _, xi, xj, _, sg, _ = bufs[b]
    for (o, n) in SUBG:
      pltpu.make_async_copy(
          x_hbm.at[db.at[pl.ds(o, n)]], xi.at[pl.ds(o, n)], sg).wait()
      pltpu.make_async_copy(
          x_hbm.at[sb.at[pl.ds(o, n)]], xj.at[pl.ds(o, n)], sg).wait()

  def compute(b):
    _, _, wb, xi, xj, _, _, _ = bufs[b]

    def edge2(e2, c2):
      for u in range(2):
        e = e2 * 2 + u
        we = wb[pl.ds(e, L)][0]
        for t in range(HB):
          sl = pl.ds(t * L, L)
          xj[e, sl] = (xj[e, sl] - xi[e, sl]) * we
      return c2

    lax.fori_loop(0, CB // 2, edge2, 0)

  def fire_writes(ci, b):
    _, _, _, xi, xj, _, _, sw = bufs[b]
    off = base + ci * CB
    pltpu.make_async_copy(xi, a1_hbm.at[pl.ds(off, CB)], sw).start()
    pltpu.make_async_copy(xj, a2_hbm.at[pl.ds(off, CB)], sw).start()

  def wait_writes(b):
    _, _, _, xi, xj, _, _, sw = bufs[b]
    pltpu.make_async_copy(xi, a1_hbm.at[pl.ds(0, CB)], sw).wait()
    pltpu.make_async_copy(xj, a2_hbm.at[pl.ds(0, CB)], sw).wait()

  fire_ids(0, 0)
  wait_ids(0)
  fire_gathers(0)
  fire_ids(1, 1)

  def pair(j, c):
    c0 = 2 * j
    wait_gathers(0)
    wait_ids(1)
    pl.when(j > 0)(lambda: wait_writes(1))
    fire_gathers(1)
    compute(0)
    fire_writes(c0, 0)
    pl.when(c0 + 2 < NCH_A)(lambda: fire_ids(c0 + 2, 0))
    wait_gathers(1)

    @pl.when(c0 + 2 < NCH_A)
    def _next():
      wait_ids(0)
      wait_writes(0)
      fire_gathers(0)

    compute(1)
    fire_writes(c0 + 1, 1)
    pl.when(c0 + 3 < NCH_A)(lambda: fire_ids(c0 + 3, 1))
    return c

  lax.fori_loop(0, NCH_A // 2, pair, 0)
  wait_writes(0)
  wait_writes(1)


def _sc_gather(x, src, dst, w):
  mesh = plsc.VectorSubcoreMesh(core_axis_name="c", subcore_axis_name="s")
  sdt = pltpu.SemaphoreType.DMA
  buf = lambda: [
      pltpu.VMEM((CB,), jnp.int32),
      pltpu.VMEM((CB,), jnp.int32),
      pltpu.VMEM((CB + L,), jnp.float32),
      pltpu.VMEM((CB, H), jnp.float32),
      pltpu.VMEM((CB, H), jnp.float32),
  ]
  f = pl.kernel(
      _gather_body,
      out_type=(
          jax.ShapeDtypeStruct((E, H), jnp.float32),
          jax.ShapeDtypeStruct((E, H), jnp.float32),
      ),
      mesh=mesh,
      compiler_params=pltpu.CompilerParams(needs_layout_passes=False),
      scratch_types=buf() + buf() + [sdt, sdt, sdt, sdt, sdt, sdt],
  )
  return f(x, src, dst, w)


def _ln_lrelu(h, g, b):
  mu = jnp.mean(h, axis=1, keepdims=True)
  var = jnp.mean((h - mu) ** 2, axis=1, keepdims=True)
  hn = (h - mu) * lax.rsqrt(var + 1e-5) * g + b
  return jnp.where(hn >= 0, hn, 0.2 * hn)


def _dot_bf16(h, w):
  return jnp.dot(h.astype(jnp.bfloat16), w.astype(jnp.bfloat16),
                 preferred_element_type=jnp.float32)


def _mlp_body(a1_ref, a2_ref, g1, b1, w1, g2, b2, w2, g3, b3, w3, o_ref):
  h = jnp.concatenate([a1_ref[...], a2_ref[...]], axis=1)
  h = _dot_bf16(_ln_lrelu(h, g1[...], b1[...]), w1[...])
  h = _dot_bf16(_ln_lrelu(h, g2[...], b2[...]), w2[...])
  h = _dot_bf16(_ln_lrelu(h, g3[...], b3[...]), w3[...])
  o_ref[...] = h


def _tc_mlp(a1, a2, g1, b1, w1, g2, b2, w2, g3, b3, w3):
  be = 2000
  nb = E // be
  full = lambda shape: pl.BlockSpec(shape, lambda i: (0, 0))
  return pl.pallas_call(
      _mlp_body,
      grid=(nb,),
      in_specs=[
          pl.BlockSpec((be, H), lambda i: (i, 0)),
          pl.BlockSpec((be, H), lambda i: (i, 0)),
          full((1, 2 * H)), full((1, 2 * H)), full((2 * H, H)),
          full((1, H)), full((1, H)), full((H, H)),
          full((1, H)), full((1, H)), full((H, H)),
      ],
      out_specs=pl.BlockSpec((be, H), lambda i: (i, 0)),
      out_shape=jax.ShapeDtypeStruct((E, H), jnp.float32),
      compiler_params=pltpu.CompilerParams(
          dimension_semantics=("arbitrary",)),
  )(a1, a2, g1.reshape(1, -1), b1.reshape(1, -1), w1,
    g2.reshape(1, -1), b2.reshape(1, -1), w2,
    g3.reshape(1, -1), b3.reshape(1, -1), w3)


def _segmax_body(h_hbm, dst_hbm, x_hbm, o_hbm,
                 dbuf0, dbuf1, cnts, locid, ldst,
                 rows0, rows1, rows2, rows3, acc, shared,
                 semd0, semd1, semr0, semr1, semr2, semr3, semst):
  sid = lax.axis_index("s")
  wid = sid * NC + lax.axis_index("c")
  lo = wid * PN
  neg_inf = jnp.full((L,), -jnp.inf, jnp.float32)

  def initr(r, c):
    for t in range(HB):
      acc[r, pl.ds(t * L, L)] = neg_inf
    return c

  lax.fori_loop(0, PN, initr, 0)

  iota = lax.iota(jnp.int32, L)

  def fire_d(ci, dbuf, semd):
    pltpu.make_async_copy(
        dst_hbm.at[pl.ds(ci * CH, CH)], dbuf, semd).start()

  def wait_d(dbuf, semd):
    pltpu.make_async_copy(dst_hbm.at[pl.ds(0, CH)], dbuf, semd).wait()

  def stage_fire(ci, b):
    pltpu.make_async_copy(
        h_hbm.at[pl.ds(ci * CH + sid * SROWS, SROWS)],
        shared.at[b].at[pl.ds(sid * SROWS, SROWS)], semst).start()

  def stage_wait():
    pltpu.make_async_copy(
        h_hbm.at[pl.ds(0, SROWS)],
        shared.at[0].at[pl.ds(0, SROWS)], semst).wait()

  def process(ci, dbuf, sb):

    def p1(k4, c):
      for u in range(4):
        k = k4 * 4 + u
        v = dbuf[pl.ds(k * L, L)]
        msk = (v >= lo) & (v < lo + PN)
        cnts[k] = jnp.sum(msk.astype(jnp.int32))
      return c

    lax.fori_loop(0, NST // 4, p1, 0)

    def p2(k2, m):
      for u in range(2):
        k = k2 * 2 + u
        v = dbuf[pl.ds(k * L, L)]
        msk = (v >= lo) & (v < lo + PN)
        gid = iota + k * L
        plsc.store_compressed(locid.at[pl.ds(m, L)], gid, mask=msk)
        plsc.store_compressed(ldst.at[pl.ds(m, L)], v - lo, mask=msk)
        m = m + cnts[k]
      return m

    m_tot = lax.fori_loop(0, NST // 2, p2, jnp.int32(0))

    dumm = jnp.full((L,), PN, jnp.int32)
    for j in range(Q // L):
      locid[pl.ds(m_tot + j * L, L)] = iota + (wid * 32 + j * L)
      ldst[pl.ds(m_tot + j * L, L)] = dumm

    nq = lax.shift_right_logical(m_tot + (Q - 1), 6)

    def fire_r(q, rows, semr):
      pltpu.make_async_copy(
          shared.at[sb].at[locid.at[pl.ds(q * Q, Q)]], rows, semr).start()

    def wait_r(rows, semr):
      pltpu.make_async_copy(
          shared.at[sb].at[locid.at[pl.ds(0, Q)]], rows, semr).wait()

    def update(q, rows):
      qb = q * Q

      def upd(i2, c2):
        for u in range(2):
          i = i2 * 2 + u
          r = ldst[pl.ds(qb + i, L)][0]
          for t in range(HB):
            sl = pl.ds(t * L, L)
            acc[r, sl] = jnp.maximum(acc[r, sl], rows[i, sl])
        return c2

      lax.fori_loop(0, Q // 2, upd, 0)

    ring = ((rows0, semr0), (rows1, semr1), (rows2, semr2), (rows3, semr3))
    NR = len(ring)

    for u in range(NR):
      pl.when(u < nq)(lambda _u=u: fire_r(_u, ring[_u][0], ring[_u][1]))

    def quad(j, c):
      for u in range(NR):
        q = j * NR + u
        rows, semr = ring[u]

        @pl.when(q < nq)
        def _go(q=q, rows=rows, semr=semr):
          wait_r(rows, semr)
          update(q, rows)
          pl.when(q + NR < nq)(lambda: fire_r(q + NR, rows, semr))

      return c

    lax.fori_loop(0, lax.shift_right_logical(nq + (NR - 1), 2), quad, 0)

  fire_d(0, dbuf0, semd0)
  stage_fire(0, 0)
  stage_wait()
  plsc.subcore_barrier()

  def cpair(c2, c):
    ci0 = 2 * c2
    wait_d(dbuf0, semd0)
    fire_d(ci0 + 1, dbuf1, semd1)
    stage_fire(ci0 + 1, 1)
    process(ci0, dbuf0, 0)
    stage_wait()
    plsc.subcore_barrier()
    wait_d(dbuf1, semd1)
    pl.when(ci0 + 2 < NCH_C)(lambda: fire_d(ci0 + 2, dbuf0, semd0))
    pl.when(ci0 + 2 < NCH_C)(lambda: stage_fire(ci0 + 2, 0))
    process(ci0 + 1, dbuf1, 1)

    @pl.when(ci0 + 2 < NCH_C)
    def _sync():
      stage_wait()
      plsc.subcore_barrier()

    return c

  lax.fori_loop(0, NCH_C // 2, cpair, 0)

  for b in range(PN // Q):
    pltpu.sync_copy(x_hbm.at[pl.ds(lo + b * Q, Q)], rows0)

    def fin(i, c, _b=b):
      for t in range(HB):
        sl = pl.ds(t * L, L)
        a = acc[_b * Q + i, sl]
        acc[_b * Q + i, sl] = jnp.where(a == -jnp.inf, 0.0, a) + rows0[i, sl]
      return c

    lax.fori_loop(0, Q, fin, 0)

  pltpu.sync_copy(acc.at[pl.ds(0, PN)], o_hbm.at[pl.ds(lo, PN)])


def _sc_segmax(h, dst, x_pad):
  mesh = plsc.VectorSubcoreMesh(core_axis_name="c", subcore_axis_name="s")
  f = pl.kernel(
      _segmax_body,
      out_type=jax.ShapeDtypeStruct((NPAD, H), jnp.float32),
      mesh=mesh,
      compiler_params=pltpu.CompilerParams(needs_layout_passes=False),
      scratch_types=[
          pltpu.VMEM((CH,), jnp.int32),
          pltpu.VMEM((CH,), jnp.int32),
          pltpu.SMEM((NST,), jnp.int32),
          pltpu.VMEM((CH + 2 * Q,), jnp.int32),
          pltpu.VMEM((CH + 2 * Q,), jnp.int32),
          pltpu.VMEM((Q, H), jnp.float32),
          pltpu.VMEM((Q, H), jnp.float32),
          pltpu.VMEM((Q, H), jnp.float32),
          pltpu.VMEM((Q, H), jnp.float32),
          pltpu.VMEM((PN + 8, H), jnp.float32),
          pltpu.VMEM_SHARED((2, CH, H), jnp.float32),
          pltpu.SemaphoreType.DMA,
          pltpu.SemaphoreType.DMA,
          pltpu.SemaphoreType.DMA,
          pltpu.SemaphoreType.DMA,
          pltpu.SemaphoreType.DMA,
          pltpu.SemaphoreType.DMA,
          pltpu.SemaphoreType.DMA,
      ],
  )
  return f(h, dst, x_pad)


def kernel(x, edge_index, edge_weight, ln1_g, ln1_b, W1,
           ln2_g, ln2_b, W2, ln3_g, ln3_b, W3):
  src = edge_index[0]
  dst = edge_index[1]
  a1, a2 = _sc_gather(x, src, dst, edge_weight)
  h = _tc_mlp(a1, a2, ln1_g, ln1_b, W1, ln2_g, ln2_b, W2, ln3_g, ln3_b, W3)
  x_pad = jnp.pad(x, ((0, NPAD - N), (0, 0)))
  out = _sc_segmax(h, dst, x_pad)
  return out[:N]


# speedup vs baseline: 1.2106x; 1.2106x over previous
import jax
import jax.numpy as jnp
from jax import lax
from jax.experimental import pallas as pl
from jax.experimental.pallas import tpu as pltpu
from jax.experimental.pallas import tpu_sc as plsc

NC = 2
NS = 16
NW = NC * NS
L = 16

N = 10000
E = 320000
H = 128
HB = H // L

PN = 320
NPAD = NW * PN

EPW = E // NW
CB = 200
NCH_A = EPW // CB
SUBG = ((0, 128), (128, CB - 128))

CH = 8000
NCH_C = E // CH
NST = CH // L
Q = 32
NR = 8


def _gather_body(x_hbm, src_hbm, dst_hbm, w_hbm, a1_hbm, a2_hbm,
                 sb0, db0, wb0, xi0, xj0, sb1, db1, wb1, xi1, xj1,
                 si0, si1, sg0, sg1, sw0, sw1):
  wid = lax.axis_index("s") * NC + lax.axis_index("c")
  base = wid * EPW

  bufs = ((sb0, db0, wb0, xi0, xj0, si0, sg0, sw0),
          (sb1, db1, wb1, xi1, xj1, si1, sg1, sw1))

  def fire_ids(ci, b):
    sb, db, wb, _, _, si, _, _ = bufs[b]
    off = base + ci * CB
    pltpu.make_async_copy(src_hbm.at[pl.ds(off, CB)], sb, si).start()
    pltpu.make_async_copy(dst_hbm.at[pl.ds(off, CB)], db, si).start()
    pltpu.make_async_copy(
        w_hbm.at[pl.ds(off, CB)], wb.at[pl.ds(0, CB)], si).start()

  def wait_ids(b):
    sb, db, wb, _, _, si, _, _ = bufs[b]
    pltpu.make_async_copy(src_hbm.at[pl.ds(0, CB)], sb, si).wait()
    pltpu.make_async_copy(dst_hbm.at[pl.ds(0, CB)], db, si).wait()
    pltpu.make_async_copy(
        w_hbm.at[pl.ds(0, CB)], wb.at[pl.ds(0, CB)], si).wait()

  def fire_gathers(b):
    sb, db, _, xi, xj, _, sg, _ = bufs[b]
    for (o, n) in SUBG:
      pltpu.make_async_copy(
          x_hbm.at[db.at[pl.ds(o, n)]], xi.at[pl.ds(o, n)], sg).start()
      pltpu.make_async_copy(
          x_hbm.at[sb.at[pl.ds(o, n)]], xj.at[pl.ds(o, n)], sg).start()

  def wait_gathers(b):
    sb, db, _, xi, xj, _, sg, _ = bufs[b]
    for (o, n) in SUBG:
      pltpu.make_async_copy(
          x_hbm.at[db.at[pl.ds(o, n)]], xi.at[pl.ds(o, n)], sg).wait()
      pltpu.make_async_copy(
          x_hbm.at[sb.at[pl.ds(o, n)]], xj.at[pl.ds(o, n)], sg).wait()

  def compute(b):
    _, _, wb, xi, xj, _, _, _ = bufs[b]

    def edge2(e2, c2):
      for u in range(2):
        e = e2 * 2 + u
        we = wb[pl.ds(e, L)][0]
        for t in range(HB):
          sl = pl.ds(t * L, L)
          xj[e, sl] = (xj[e, sl] - xi[e, sl]) * we
      return c2

    lax.fori_loop(0, CB // 2, edge2, 0)

  def fire_writes(ci, b):
    _, _, _, xi, xj, _, _, sw = bufs[b]
    off = base + ci * CB
    pltpu.make_async_copy(xi, a1_hbm.at[pl.ds(off, CB)], sw).start()
    pltpu.make_async_copy(xj, a2_hbm.at[pl.ds(off, CB)], sw).start()

  def wait_writes(b):
    _, _, _, xi, xj, _, _, sw = bufs[b]
    pltpu.make_async_copy(xi, a1_hbm.at[pl.ds(0, CB)], sw).wait()
    pltpu.make_async_copy(xj, a2_hbm.at[pl.ds(0, CB)], sw).wait()

  fire_ids(0, 0)
  wait_ids(0)
  fire_gathers(0)
  fire_ids(1, 1)

  def pair(j, c):
    c0 = 2 * j
    wait_gathers(0)
    wait_ids(1)
    pl.when(j > 0)(lambda: wait_writes(1))
    fire_gathers(1)
    compute(0)
    fire_writes(c0, 0)
    pl.when(c0 + 2 < NCH_A)(lambda: fire_ids(c0 + 2, 0))
    wait_gathers(1)

    @pl.when(c0 + 2 < NCH_A)
    def _next():
      wait_ids(0)
      wait_writes(0)
      fire_gathers(0)

    compute(1)
    fire_writes(c0 + 1, 1)
    pl.when(c0 + 3 < NCH_A)(lambda: fire_ids(c0 + 3, 1))
    return c

  lax.fori_loop(0, NCH_A // 2, pair, 0)
  wait_writes(0)
  wait_writes(1)


def _sc_gather(x, src, dst, w):
  mesh = plsc.VectorSubcoreMesh(core_axis_name="c", subcore_axis_name="s")
  sdt = pltpu.SemaphoreType.DMA
  buf = lambda: [
      pltpu.VMEM((CB,), jnp.int32),
      pltpu.VMEM((CB,), jnp.int32),
      pltpu.VMEM((CB + L,), jnp.float32),
      pltpu.VMEM((CB, H), jnp.float32),
      pltpu.VMEM((CB, H), jnp.float32),
  ]
  f = pl.kernel(
      _gather_body,
      out_type=(
          jax.ShapeDtypeStruct((E, H), jnp.float32),
          jax.ShapeDtypeStruct((E, H), jnp.float32),
      ),
      mesh=mesh,
      compiler_params=pltpu.CompilerParams(needs_layout_passes=False),
      scratch_types=buf() + buf() + [sdt, sdt, sdt, sdt, sdt, sdt],
  )
  return f(x, src, dst, w)


def _ln_lrelu(h, g, b):
  mu = jnp.mean(h, axis=1, keepdims=True)
  var = jnp.mean((h - mu) ** 2, axis=1, keepdims=True)
  hn = (h - mu) * lax.rsqrt(var + 1e-5) * g + b
  return jnp.where(hn >= 0, hn, 0.2 * hn)


def _dot_bf16(h, w):
  return jnp.dot(h.astype(jnp.bfloat16), w.astype(jnp.bfloat16),
                 preferred_element_type=jnp.float32)


def _mlp_body(a1_ref, a2_ref, g1, b1, w1, g2, b2, w2, g3, b3, w3, o_ref):
  h = jnp.concatenate([a1_ref[...], a2_ref[...]], axis=1)
  h = _dot_bf16(_ln_lrelu(h, g1[...], b1[...]), w1[...])
  h = _dot_bf16(_ln_lrelu(h, g2[...], b2[...]), w2[...])
  h = _dot_bf16(_ln_lrelu(h, g3[...], b3[...]), w3[...])
  o_ref[...] = h


def _tc_mlp(a1, a2, g1, b1, w1, g2, b2, w2, g3, b3, w3):
  be = 2000
  nb = E // be
  full = lambda shape: pl.BlockSpec(shape, lambda i: (0, 0))
  return pl.pallas_call(
      _mlp_body,
      grid=(nb,),
      in_specs=[
          pl.BlockSpec((be, H), lambda i: (i, 0)),
          pl.BlockSpec((be, H), lambda i: (i, 0)),
          full((1, 2 * H)), full((1, 2 * H)), full((2 * H, H)),
          full((1, H)), full((1, H)), full((H, H)),
          full((1, H)), full((1, H)), full((H, H)),
      ],
      out_specs=pl.BlockSpec((be, H), lambda i: (i, 0)),
      out_shape=jax.ShapeDtypeStruct((E, H), jnp.float32),
      compiler_params=pltpu.CompilerParams(
          dimension_semantics=("arbitrary",)),
  )(a1, a2, g1.reshape(1, -1), b1.reshape(1, -1), w1,
    g2.reshape(1, -1), b2.reshape(1, -1), w2,
    g3.reshape(1, -1), b3.reshape(1, -1), w3)


def _segmax_body(h_hbm, dst_hbm, x_hbm, o_hbm,
                 dbuf0, dbuf1, cnts, locid, ldst, rowsl, acc,
                 semd0, semd1, *semr):
  wid = lax.axis_index("s") * NC + lax.axis_index("c")
  lo = wid * PN
  neg_inf = jnp.full((L,), -jnp.inf, jnp.float32)

  def initr(r, c):
    for t in range(HB):
      acc[r, pl.ds(t * L, L)] = neg_inf
    return c

  lax.fori_loop(0, PN, initr, 0)

  iota = lax.iota(jnp.int32, L)

  def fire_d(ci, dbuf, semd):
    pltpu.make_async_copy(
        dst_hbm.at[pl.ds(ci * CH, CH)], dbuf, semd).start()

  def wait_d(dbuf, semd):
    pltpu.make_async_copy(dst_hbm.at[pl.ds(0, CH)], dbuf, semd).wait()

  def process(ci, dbuf):
    cbase = ci * CH

    def p1(k4, c):
      for u in range(4):
        k = k4 * 4 + u
        v = dbuf[pl.ds(k * L, L)]
        msk = (v >= lo) & (v < lo + PN)
        cnts[k] = jnp.sum(msk.astype(jnp.int32))
      return c

    lax.fori_loop(0, NST // 4, p1, 0)

    def p2(k2, m):
      for u in range(2):
        k = k2 * 2 + u
        v = dbuf[pl.ds(k * L, L)]
        msk = (v >= lo) & (v < lo + PN)
        gid = iota + (cbase + k * L)
        plsc.store_compressed(locid.at[pl.ds(m, L)], gid, mask=msk)
        plsc.store_compressed(ldst.at[pl.ds(m, L)], v - lo, mask=msk)
        m = m + cnts[k]
      return m

    m_tot = lax.fori_loop(0, NST // 2, p2, jnp.int32(0))

    dumm = jnp.full((L,), PN, jnp.int32)
    for j in range(Q // L):
      locid[pl.ds(m_tot + j * L, L)] = iota + (wid * 256 + j * L)
      ldst[pl.ds(m_tot + j * L, L)] = dumm

    nq = lax.shift_right_logical(m_tot + (Q - 1), 5)

    def fire_r(q, b):
      pltpu.make_async_copy(
          h_hbm.at[locid.at[pl.ds(q * Q, Q)]],
          rowsl.at[b], semr[b]).start()

    def wait_r(b):
      pltpu.make_async_copy(
          h_hbm.at[locid.at[pl.ds(0, Q)]],
          rowsl.at[b], semr[b]).wait()

    def update(q, b):
      qb = q * Q

      def upd(i2, c2):
        for u in range(2):
          i = i2 * 2 + u
          r = ldst[pl.ds(qb + i, L)][0]
          for t in range(HB):
            sl = pl.ds(t * L, L)
            acc[r, sl] = jnp.maximum(acc[r, sl], rowsl[b, i, sl])
        return c2

      lax.fori_loop(0, Q // 2, upd, 0)

    for u in range(NR):
      pl.when(u < nq)(lambda _u=u: fire_r(_u, _u))

    def ringstep(j, c):
      for u in range(NR):
        q = j * NR + u

        @pl.when(q < nq)
        def _go(q=q, u=u):
          wait_r(u)
          update(q, u)
          pl.when(q + NR < nq)(lambda: fire_r(q + NR, u))

      return c

    lax.fori_loop(0, lax.shift_right_logical(nq + (NR - 1), 3), ringstep, 0)

  fire_d(0, dbuf0, semd0)

  def cpair(c2, c):
    ci0 = 2 * c2
    wait_d(dbuf0, semd0)
    fire_d(ci0 + 1, dbuf1, semd1)
    process(ci0, dbuf0)
    wait_d(dbuf1, semd1)
    pl.when(ci0 + 2 < NCH_C)(lambda: fire_d(ci0 + 2, dbuf0, semd0))
    process(ci0 + 1, dbuf1)
    return c

  lax.fori_loop(0, NCH_C // 2, cpair, 0)

  for b in range(PN // Q):
    pltpu.sync_copy(x_hbm.at[pl.ds(lo + b * Q, Q)], rowsl.at[0])

    def fin(i, c, _b=b):
      for t in range(HB):
        sl = pl.ds(t * L, L)
        a = acc[_b * Q + i, sl]
        acc[_b * Q + i, sl] = jnp.where(a == -jnp.inf, 0.0, a) + rowsl[0, i, sl]
      return c

    lax.fori_loop(0, Q, fin, 0)

  pltpu.sync_copy(acc.at[pl.ds(0, PN)], o_hbm.at[pl.ds(lo, PN)])


def _sc_segmax(h, dst, x_pad):
  mesh = plsc.VectorSubcoreMesh(core_axis_name="c", subcore_axis_name="s")
  f = pl.kernel(
      _segmax_body,
      out_type=jax.ShapeDtypeStruct((NPAD, H), jnp.float32),
      mesh=mesh,
      compiler_params=pltpu.CompilerParams(needs_layout_passes=False),
      scratch_types=[
          pltpu.VMEM((CH,), jnp.int32),
          pltpu.VMEM((CH,), jnp.int32),
          pltpu.SMEM((NST,), jnp.int32),
          pltpu.VMEM((CH + 2 * Q,), jnp.int32),
          pltpu.VMEM((CH + 2 * Q,), jnp.int32),
          pltpu.VMEM((NR, Q, H), jnp.float32),
          pltpu.VMEM((PN + 8, H), jnp.float32),
      ] + [pltpu.SemaphoreType.DMA] * (2 + NR),
  )
  return f(h, dst, x_pad)


def kernel(x, edge_index, edge_weight, ln1_g, ln1_b, W1,
           ln2_g, ln2_b, W2, ln3_g, ln3_b, W3):
  src = edge_index[0]
  dst = edge_index[1]
  a1, a2 = _sc_gather(x, src, dst, edge_weight)
  h = _tc_mlp(a1, a2, ln1_g, ln1_b, W1, ln2_g, ln2_b, W2, ln3_g, ln3_b, W3)
  x_pad = jnp.pad(x, ((0, NPAD - N), (0, 0)))
  out = _sc_segmax(h, dst, x_pad)
  return out[:N]
